# baseline (speedup 1.0000x reference)
import functools

import jax
import jax.numpy as jnp
from jax import lax
from jax.experimental import pallas as pl
from jax.experimental.pallas import tpu as pltpu
from jax.experimental.pallas import tpu_sc as plsc

NUM_HARMONICS = 64
NUM_ENTRIES = 100000
BATCH = 16384

NC, NS, L = 2, 16, 16
NW = NC * NS
BPW = BATCH // NW
CHUNK = 128
NCHUNK = BPW // CHUNK
FUSED = 2 * NUM_HARMONICS

_SCALE = float(NUM_ENTRIES - 1)


def _body(dial_hbm, table_hbm, out_hbm,
          dial_v, f_lo_v, f_hi_v, alpha_v, poff_v, qoff_v,
          buf_a, buf_b, out_c, sems, out_sems):
    wid = lax.axis_index("s") * NC + lax.axis_index("c")
    base = wid * BPW

    pltpu.sync_copy(dial_hbm.at[pl.ds(base, BPW)], dial_v)

    for j in range(NCHUNK):
        for k in range(CHUNK // L):
            off = k * L
            d = dial_v[pl.ds(j * CHUNK + off, L)]
            idx_f = d * _SCALE
            lo = idx_f.astype(jnp.int32)
            lo = jnp.minimum(jnp.maximum(lo, 0), NUM_ENTRIES - 2)
            alpha = idx_f - lo.astype(jnp.float32)
            hi = lo + 1
            f_lo_v[j, pl.ds(off, L)] = lax.shift_right_logical(lo, 1)
            f_hi_v[j, pl.ds(off, L)] = lax.shift_right_logical(hi, 1)
            sl = pl.ds(j * CHUNK + off, L)
            alpha_v[sl] = alpha
            poff_v[sl] = (lo & 1) * NUM_HARMONICS
            qoff_v[sl] = (hi & 1) * NUM_HARMONICS

    def fire(j):
        s = j & 1
        return (pltpu.async_copy(table_hbm.at[f_lo_v.at[j]],
                                 buf_a.at[s], sems[2 * s]),
                pltpu.async_copy(table_hbm.at[f_hi_v.at[j]],
                                 buf_b.at[s], sems[2 * s + 1]))

    inflight = {0: fire(0)}
    out_inflight = {}
    for j in range(NCHUNK):
        if j + 1 < NCHUNK:
            inflight[j + 1] = fire(j + 1)
        ca, cb = inflight.pop(j)
        ca.wait()
        cb.wait()
        s = j & 1
        if j - 2 in out_inflight:
            out_inflight.pop(j - 2).wait()

        def group_body(g, carry, s=s, j=j):
            gbase = j * CHUNK + g * L
            av = alpha_v[pl.ds(gbase, L)]
            pv = poff_v[pl.ds(gbase, L)]
            qv = qoff_v[pl.ds(gbase, L)]
            for k in range(L):
                a = av[k]
                po = pv[k]
                qo = qv[k]
                r = g * L + k
                for c in range(NUM_HARMONICS // L):
                    x = buf_a[s, r, pl.ds(po + c * L, L)]
                    y = buf_b[s, r, pl.ds(qo + c * L, L)]
                    out_c[s, r, pl.ds(c * L, L)] = x + a * (y - x)
            return carry

        lax.fori_loop(0, CHUNK // L, group_body, 0)
        out_inflight[j] = pltpu.async_copy(
            out_c.at[s], out_hbm.at[pl.ds(base + j * CHUNK, CHUNK)],
            out_sems[s])

    for j in sorted(out_inflight):
        out_inflight[j].wait()


@jax.jit
def _run(dial_flat, table_fused):
    mapper = pl.kernel(
        _body,
        out_type=jax.ShapeDtypeStruct((BATCH, NUM_HARMONICS), jnp.float32),
        mesh=plsc.VectorSubcoreMesh(
            core_axis_name="c", subcore_axis_name="s",
            num_cores=NC, num_subcores=NS),
        scratch_types=[
            pltpu.VMEM((BPW,), jnp.float32),
            pltpu.VMEM((NCHUNK, CHUNK), jnp.int32),
            pltpu.VMEM((NCHUNK, CHUNK), jnp.int32),
            pltpu.VMEM((BPW,), jnp.float32),
            pltpu.VMEM((BPW,), jnp.int32),
            pltpu.VMEM((BPW,), jnp.int32),
            pltpu.VMEM((2, CHUNK, FUSED), jnp.float32),
            pltpu.VMEM((2, CHUNK, FUSED), jnp.float32),
            pltpu.VMEM((2, CHUNK, NUM_HARMONICS), jnp.float32),
            [pltpu.SemaphoreType.DMA] * 4,
            [pltpu.SemaphoreType.DMA] * 2,
        ],
    )
    return mapper(dial_flat, table_fused)


def kernel(wave_dial_normalized, table):
    return _run(wave_dial_normalized.reshape(-1),
                table.reshape(NUM_ENTRIES // 2, FUSED))
